# baseline (speedup 1.0000x reference)
import functools

import jax
import jax.numpy as jnp
import numpy as np
from jax import lax
from jax.experimental import pallas as pl
from jax.experimental.pallas import tpu as pltpu
from jax.experimental.pallas import tpu_sc as plsc

_CARD = 100000
_F = 26
_D = 32
_B = 16384
_NW = 32
_WB = _B // _NW
_GB = 128
_NQ = _WB // _GB


def _make_kernel():
    mesh = plsc.VectorSubcoreMesh(core_axis_name="c", subcore_axis_name="s")

    @functools.partial(
        pl.kernel,
        mesh=mesh,
        out_type=jax.ShapeDtypeStruct((_F * _D, _B), jnp.float32),
        compiler_params=pltpu.CompilerParams(
            use_tc_tiling_on_sc=False, needs_layout_passes=False
        ),
        scratch_types=[
            pltpu.VMEM((_WB,), jnp.int32),
            pltpu.VMEM((_WB,), jnp.int32),
            pltpu.VMEM((_NQ, _GB), jnp.int32),
            pltpu.VMEM((_D, _WB), jnp.float32),
            pltpu.VMEM((_D, 16), jnp.float32),
            pltpu.VMEM((_F * _D,), jnp.float32),
            pltpu.VMEM((_D,), jnp.float32),
            pltpu.SemaphoreType.DMA,
        ],
    )
    def k(xt_hbm, mt_hbm, tt_hbm, bias_hbm, mw_hbm, out_hbm,
          x_v, m_v, idx_v, g_v, a_v, bias_v, mw_v, sem):
        nc = 2
        wid = lax.axis_index("s") * nc + lax.axis_index("c")
        b0 = wid * _WB
        pltpu.sync_copy(bias_hbm, bias_v)
        pltpu.sync_copy(mw_hbm, mw_v)
        mw0 = mw_v[pl.ds(0, 16)]
        mw1 = mw_v[pl.ds(16, 16)]
        cvec0 = lax.iota(jnp.int32, 16)
        cvec1 = cvec0 + 16

        def f_body(f, carry):
            src0 = f * _B + b0
            pltpu.sync_copy(xt_hbm.at[pl.ds(src0, _WB)], x_v)
            pltpu.sync_copy(mt_hbm.at[pl.ds(src0, _WB)], m_v)
            off = f * _CARD
            for j in range(_WB // 16):
                q, kk = j // (_GB // 16), j % (_GB // 16)
                idx_v[q, pl.ds(kk * 16, 16)] = x_v[pl.ds(j * 16, 16)] + off

            bv0 = bias_v[pl.ds(f * _D, 16)]
            bv1 = bias_v[pl.ds(f * _D + 16, 16)]
            for c in range(_D):
                bfc = (bv0 if c < 16 else bv1)[c % 16]
                a_v[c, pl.ds(0, 16)] = jnp.broadcast_to(bfc, (16,))

            for c in range(_D):
                for q in range(_NQ):
                    pltpu.async_copy(
                        tt_hbm.at[c].at[idx_v.at[q]],
                        g_v.at[c, pl.ds(q * _GB, _GB)],
                        sem,
                    )
            pltpu.make_async_copy(
                out_hbm.at[pl.ds(0, _D), pl.ds(0, _WB)], g_v, sem
            ).wait()

            def c_body(c, c2):
                a16 = a_v[c, pl.ds(0, 16)]
                for j in range(_WB // 16):
                    sl = pl.ds(j * 16, 16)
                    g_v[c, sl] = g_v[c, sl] + a16
                return c2

            lax.fori_loop(0, _D, c_body, 0)

            def m_body(j, c2):
                m16 = m_v[pl.ds(j * 16, 16)]
                for i in range(16):
                    @pl.when(m16[i] != 0)
                    def _():
                        bvec = jnp.broadcast_to(j * 16 + i, (16,))
                        plsc.store_scatter(g_v, [cvec0, bvec], mw0)
                        plsc.store_scatter(g_v, [cvec1, bvec], mw1)
                return c2

            lax.fori_loop(0, _WB // 16, m_body, 0)

            pltpu.sync_copy(
                g_v, out_hbm.at[pl.ds(f * _D, _D), pl.ds(b0, _WB)]
            )
            return carry

        lax.fori_loop(0, _F, f_body, 0)

    return k


_KERNEL = _make_kernel()


@jax.jit
def kernel(x, mask, table, bias, mask_weight):
    xt = jnp.transpose(x).reshape(-1)
    mt = jnp.transpose(mask).astype(jnp.int32).reshape(-1)
    tt = jnp.transpose(table)
    bias_f = bias.reshape(-1)
    mw_f = mask_weight.reshape(-1)
    out_t = _KERNEL(xt, mt, tt, bias_f, mw_f)
    return jnp.transpose(out_t.reshape(_F, _D, _B), (2, 0, 1))


# speedup vs baseline: 2.9478x; 2.9478x over previous
import functools

import jax
import jax.numpy as jnp
import numpy as np
from jax import lax
from jax.experimental import pallas as pl
from jax.experimental.pallas import tpu as pltpu
from jax.experimental.pallas import tpu_sc as plsc

_CARD = 100000
_F = 26
_D = 32
_B = 16384
_T = 2600000
_NW = 32
_WB = _B // _NW
_CB = 64
_C = _CB * _F
_GB = 128
_NQ = _C // _GB
_NCH = _WB // _CB

_TQ = 512
_NQUAD = 20312 * 128 // _TQ
_G1 = 159
_REM = 64


def _build_patterns():
    e = np.arange(_C, dtype=np.int32)
    f = e % _F
    bl = e // _F
    return (
        f.reshape(_NQ, _GB),
        bl.reshape(_NQ, _GB),
        (f * _CARD).reshape(_NQ, _GB),
    )


_FPAT, _BLPAT, _OFFPAT = _build_patterns()


def _make_relayout():
    mesh = plsc.VectorSubcoreMesh(core_axis_name="c", subcore_axis_name="s")

    @functools.partial(
        pl.kernel,
        mesh=mesh,
        out_type=jax.ShapeDtypeStruct((_T // 4, 128), jnp.float32),
        compiler_params=pltpu.CompilerParams(
            use_tc_tiling_on_sc=True, needs_layout_passes=False
        ),
        scratch_types=[
            pltpu.VMEM((_D, _TQ), jnp.float32),
            pltpu.VMEM((_TQ // 4, 128), jnp.float32),
        ],
    )
    def k1(tt_hbm, tail_hbm, t4_hbm, c_v, r_v):
        nc = 2
        wid = lax.axis_index("s") * nc + lax.axis_index("c")
        iota = lax.iota(jnp.int32, 16)

        def quad_body(g, carry):
            q4 = g * _NW + wid

            @pl.when(q4 < _NQUAD)
            def _():
                r0 = q4 * _TQ
                pltpu.sync_copy(tt_hbm.at[:, pl.ds(r0, _TQ)], c_v)

                def c_body(c, c2):
                    for rb in range(_TQ // 16):
                        rv = rb * 16 + iota
                        ridx = lax.shift_right_logical(rv, 2)
                        jidx = lax.shift_left(
                            lax.bitwise_and(rv, 3), 5) + c
                        v16 = c_v[c, pl.ds(rb * 16, 16)]
                        plsc.store_scatter(r_v, [ridx, jidx], v16)
                    return c2

                lax.fori_loop(0, _D, c_body, 0)
                pltpu.sync_copy(r_v, t4_hbm.at[pl.ds(q4 * (_TQ // 4), _TQ // 4), :])
            return carry

        lax.fori_loop(0, _G1, quad_body, 0)

        @pl.when(wid == 0)
        def _():
            pltpu.sync_copy(tail_hbm, r_v.at[pl.ds(0, _REM // 4), :])
            pltpu.sync_copy(r_v.at[pl.ds(0, _REM // 4), :],
                            t4_hbm.at[pl.ds(_NQUAD * (_TQ // 4), _REM // 4), :])

    return k1


def _make_gather():
    mesh = plsc.VectorSubcoreMesh(core_axis_name="c", subcore_axis_name="s")

    @functools.partial(
        pl.kernel,
        mesh=mesh,
        out_type=jax.ShapeDtypeStruct((_F * _D, _B), jnp.float32),
        compiler_params=pltpu.CompilerParams(
            use_tc_tiling_on_sc=False, needs_layout_passes=False
        ),
        scratch_types=[
            pltpu.VMEM((_F, _CB), jnp.int32),
            pltpu.VMEM((_F, _CB), jnp.int32),
            pltpu.VMEM((_F, _CB), jnp.float32),
            pltpu.VMEM((_NQ, _GB), jnp.int32),
            pltpu.VMEM((_C, _D), jnp.float32),
            pltpu.VMEM((_F * _D, _CB), jnp.float32),
            pltpu.VMEM((_NQ, _GB), jnp.int32),
            pltpu.VMEM((_NQ, _GB), jnp.int32),
            pltpu.VMEM((_NQ, _GB), jnp.int32),
            pltpu.VMEM((_F * _D,), jnp.float32),
            pltpu.VMEM((_D,), jnp.float32),
            pltpu.SemaphoreType.DMA,
        ],
    )
    def k2(xt_hbm, mt_hbm, tl_hbm, bias_hbm, mw_hbm, fp_hbm, blp_hbm,
           op_hbm, out_hbm,
           x_v, m_v, k_v, idx_v, g_v, t_v, fp_v, blp_v, op_v, bias_v,
           mw_v, sem):
        nc = 2
        wid = lax.axis_index("s") * nc + lax.axis_index("c")
        b0 = wid * _WB
        pltpu.sync_copy(bias_hbm, bias_v)
        pltpu.sync_copy(mw_hbm, mw_v)
        pltpu.sync_copy(fp_hbm, fp_v)
        pltpu.sync_copy(blp_hbm, blp_v)
        pltpu.sync_copy(op_hbm, op_v)
        mw0 = mw_v[pl.ds(0, 16)]
        mw1 = mw_v[pl.ds(16, 16)]
        iota = lax.iota(jnp.int32, 16)

        def chunk_body(ch, carry):
            bc = b0 + ch * _CB
            pltpu.sync_copy(xt_hbm.at[:, pl.ds(bc, _CB)], x_v)
            pltpu.sync_copy(mt_hbm.at[:, pl.ds(bc, _CB)], m_v)

            def keep_body(f, c2):
                for v in range(_CB // 16):
                    sl = pl.ds(v * 16, 16)
                    k_v[f, sl] = (
                        jnp.float32(1) - m_v[f, sl].astype(jnp.float32)
                    )
                return c2

            lax.fori_loop(0, _F, keep_body, 0)

            def idx_body(q, c2):
                for kk in range(_GB // 16):
                    sl = pl.ds(kk * 16, 16)
                    f16 = fp_v[q, sl]
                    bl16 = blp_v[q, sl]
                    xv = plsc.load_gather(x_v, [f16, bl16])
                    idx_v[q, sl] = xv + op_v[q, sl]
                return c2

            lax.fori_loop(0, _NQ, idx_body, 0)

            for q in range(_NQ):
                pltpu.async_copy(
                    tl_hbm.at[idx_v.at[q]],
                    g_v.at[pl.ds(q * _GB, _GB)],
                    sem,
                )
            pltpu.make_async_copy(
                tl_hbm.at[pl.ds(0, _C), :], g_v, sem
            ).wait()

            def f_body(f, c2):
                bv0 = bias_v[pl.ds(f * _D, 16)]
                bv1 = bias_v[pl.ds(f * _D + 16, 16)]
                krow = [k_v[f, pl.ds(v * 16, 16)] for v in range(_CB // 16)]
                erow = [
                    (v * 16 + iota) * _F + f for v in range(_CB // 16)
                ]
                for c in range(_D):
                    a = (bv0 if c < 16 else bv1)[c % 16]
                    w = (mw0 if c < 16 else mw1)[c % 16]
                    p = a - w
                    c16 = jnp.broadcast_to(jnp.int32(c), (16,))
                    row = f * _D + c
                    for v in range(_CB // 16):
                        g16 = plsc.load_gather(g_v, [erow[v], c16])
                        t_v[row, pl.ds(v * 16, 16)] = (g16 + p) * krow[v] + w
                return c2

            lax.fori_loop(0, _F, f_body, 0)
            pltpu.sync_copy(t_v, out_hbm.at[:, pl.ds(bc, _CB)])
            return carry

        lax.fori_loop(0, _NCH, chunk_body, 0)

    return k2


_K1 = _make_relayout()
_K2 = _make_gather()


@jax.jit
def kernel(x, mask, table, bias, mask_weight):
    tt = jnp.transpose(table)
    tail4 = table[_NQUAD * _TQ:].reshape(_REM // 4, 128)
    t4 = _K1(tt, tail4)
    tlin = t4.reshape(_T, _D)
    xt = jnp.transpose(x)
    mt = jnp.transpose(mask).astype(jnp.int32)
    out2d = _K2(
        xt, mt, tlin, bias.reshape(-1), mask_weight.reshape(-1),
        jnp.asarray(_FPAT), jnp.asarray(_BLPAT), jnp.asarray(_OFFPAT),
    )
    return jnp.transpose(out2d.reshape(_F, _D, _B), (2, 0, 1))


# speedup vs baseline: 4.5749x; 1.5520x over previous
import functools

import jax
import jax.numpy as jnp
import numpy as np
from jax import lax
from jax.experimental import pallas as pl
from jax.experimental.pallas import tpu as pltpu
from jax.experimental.pallas import tpu_sc as plsc

_CARD = 100000
_F = 26
_D = 32
_B = 16384
_T = 2600000
_NW = 32
_WB = _B // _NW
_CB = 64
_C = _CB * _F
_GB = 128
_NQ = _C // _GB
_NCH = _WB // _CB

_TQ = 512
_NQUAD = 20312 * 128 // _TQ
_G1 = 80
_REM = 64


def _build_patterns():
    e = np.arange(_C, dtype=np.int32)
    f = e % _F
    bl = e // _F
    return (
        f.reshape(_NQ, _GB),
        bl.reshape(_NQ, _GB),
        (f * _CARD).reshape(_NQ, _GB),
    )


_FPAT, _BLPAT, _OFFPAT = _build_patterns()


def _make_relayout():
    mesh = plsc.VectorSubcoreMesh(core_axis_name="c", subcore_axis_name="s")

    @functools.partial(
        pl.kernel,
        mesh=mesh,
        out_type=jax.ShapeDtypeStruct((_T // 4, 128), jnp.float32),
        compiler_params=pltpu.CompilerParams(
            use_tc_tiling_on_sc=True, needs_layout_passes=False
        ),
        scratch_types=[
            pltpu.VMEM((_D, _TQ), jnp.float32),
            pltpu.VMEM((_D, _TQ), jnp.float32),
            pltpu.VMEM((_TQ // 4, 128), jnp.float32),
            pltpu.SemaphoreType.DMA,
            pltpu.SemaphoreType.DMA,
        ],
    )
    def k1(tt_hbm, tail_hbm, t4_hbm, ca_v, cb_v, r_v, sa, sb):
        nc = 2
        wid = lax.axis_index("s") * nc + lax.axis_index("c")
        iota = lax.iota(jnp.int32, 16)
        iotashr = lax.shift_right_logical(iota, 2)
        jbase = lax.shift_left(lax.bitwise_and(iota, 3), 5)

        def fire(q4, buf, sem):
            pltpu.async_copy(tt_hbm.at[:, pl.ds(q4 * _TQ, _TQ)], buf, sem)

        def process(g, buf, sem, nbuf, nsem):
            q4 = g * _NW + wid

            @pl.when(q4 < _NQUAD)
            def _():
                pltpu.make_async_copy(
                    tt_hbm.at[:, pl.ds(0, _TQ)], buf, sem
                ).wait()
                qn = (g + 1) * _NW + wid

                @pl.when(qn < _NQUAD)
                def _():
                    fire(qn, nbuf, nsem)

                @plsc.parallel_loop(0, _D, unroll=2)
                def c_body(c):
                    jidx = jbase + c
                    for rb in range(_TQ // 16):
                        ridx = iotashr + rb * 4
                        plsc.store_scatter(
                            r_v, [ridx, jidx], buf[c, pl.ds(rb * 16, 16)]
                        )

                pltpu.sync_copy(
                    r_v, t4_hbm.at[pl.ds(q4 * (_TQ // 4), _TQ // 4), :]
                )

        fire(wid, ca_v, sa)

        def pair_body(g2, carry):
            process(g2 * 2, ca_v, sa, cb_v, sb)
            process(g2 * 2 + 1, cb_v, sb, ca_v, sa)
            return carry

        lax.fori_loop(0, _G1, pair_body, 0)

        @pl.when(wid == 0)
        def _():
            pltpu.sync_copy(tail_hbm, r_v.at[pl.ds(0, _REM // 4), :])
            pltpu.sync_copy(r_v.at[pl.ds(0, _REM // 4), :],
                            t4_hbm.at[pl.ds(_NQUAD * (_TQ // 4), _REM // 4), :])

    return k1


def _make_gather():
    mesh = plsc.VectorSubcoreMesh(core_axis_name="c", subcore_axis_name="s")

    @functools.partial(
        pl.kernel,
        mesh=mesh,
        out_type=jax.ShapeDtypeStruct((_F * _D, _B), jnp.float32),
        compiler_params=pltpu.CompilerParams(
            use_tc_tiling_on_sc=False, needs_layout_passes=False
        ),
        scratch_types=[
            pltpu.VMEM((_F, _CB), jnp.int32),
            pltpu.VMEM((_F, _CB), jnp.int32),
            pltpu.VMEM((_NQ, _GB), jnp.int32),
            pltpu.VMEM((_C, _D), jnp.float32),
            pltpu.VMEM((_F * _D, _CB), jnp.float32),
            pltpu.VMEM((_NQ, _GB), jnp.int32),
            pltpu.VMEM((_NQ, _GB), jnp.int32),
            pltpu.VMEM((_NQ, _GB), jnp.int32),
            pltpu.VMEM((_F * _D,), jnp.float32),
            pltpu.VMEM((_D,), jnp.float32),
            pltpu.SemaphoreType.DMA,
            pltpu.SemaphoreType.DMA,
        ],
    )
    def k2(xt_hbm, mt_hbm, tl_hbm, bias_hbm, mw_hbm, fp_hbm, blp_hbm,
           op_hbm, out_hbm,
           x_v, m_v, idx_v, g_v, t_v, fp_v, blp_v, op_v, bias_v,
           mw_v, sem, so):
        nc = 2
        wid = lax.axis_index("s") * nc + lax.axis_index("c")
        b0 = wid * _WB
        pltpu.sync_copy(bias_hbm, bias_v)
        pltpu.sync_copy(mw_hbm, mw_v)
        pltpu.sync_copy(fp_hbm, fp_v)
        pltpu.sync_copy(blp_hbm, blp_v)
        pltpu.sync_copy(op_hbm, op_v)
        mw0 = mw_v[pl.ds(0, 16)]
        mw1 = mw_v[pl.ds(16, 16)]
        iota = lax.iota(jnp.int32, 16)

        def chunk_body(ch, carry):
            bc = b0 + ch * _CB
            pltpu.sync_copy(xt_hbm.at[:, pl.ds(bc, _CB)], x_v)
            pltpu.sync_copy(mt_hbm.at[:, pl.ds(bc, _CB)], m_v)

            @plsc.parallel_loop(0, _NQ)
            def idx_body(q):
                for kk in range(_GB // 16):
                    sl = pl.ds(kk * 16, 16)
                    f16 = fp_v[q, sl]
                    bl16 = blp_v[q, sl]
                    xv = plsc.load_gather(x_v, [f16, bl16])
                    idx_v[q, sl] = xv + op_v[q, sl]

            for q in range(_NQ):
                pltpu.async_copy(
                    tl_hbm.at[idx_v.at[q]],
                    g_v.at[pl.ds(q * _GB, _GB)],
                    sem,
                )
            pltpu.make_async_copy(
                tl_hbm.at[pl.ds(0, _C), :], g_v, sem
            ).wait()

            @pl.when(ch > 0)
            def _():
                pltpu.make_async_copy(
                    out_hbm.at[:, pl.ds(0, _CB)], t_v, so
                ).wait()

            @plsc.parallel_loop(0, _F)
            def f_body(f):
                bv0 = bias_v[pl.ds(f * _D, 16)]
                bv1 = bias_v[pl.ds(f * _D + 16, 16)]
                krow = [
                    jnp.float32(1)
                    - m_v[f, pl.ds(v * 16, 16)].astype(jnp.float32)
                    for v in range(_CB // 16)
                ]
                erow = [
                    (v * 16 + iota) * _F + f for v in range(_CB // 16)
                ]
                for c in range(_D):
                    a = (bv0 if c < 16 else bv1)[c % 16]
                    w = (mw0 if c < 16 else mw1)[c % 16]
                    p = a - w
                    c16 = jnp.broadcast_to(jnp.int32(c), (16,))
                    row = f * _D + c
                    for v in range(_CB // 16):
                        g16 = plsc.load_gather(g_v, [erow[v], c16])
                        t_v[row, pl.ds(v * 16, 16)] = (g16 + p) * krow[v] + w

            pltpu.async_copy(t_v, out_hbm.at[:, pl.ds(bc, _CB)], so)
            return carry

        lax.fori_loop(0, _NCH, chunk_body, 0)
        pltpu.make_async_copy(out_hbm.at[:, pl.ds(0, _CB)], t_v, so).wait()

    return k2


_K1 = _make_relayout()
_K2 = _make_gather()


@jax.jit
def kernel(x, mask, table, bias, mask_weight):
    tt = jnp.transpose(table)
    tail4 = table[_NQUAD * _TQ:].reshape(_REM // 4, 128)
    t4 = _K1(tt, tail4)
    tlin = t4.reshape(_T, _D)
    xt = jnp.transpose(x)
    mt = jnp.transpose(mask).astype(jnp.int32)
    out2d = _K2(
        xt, mt, tlin, bias.reshape(-1), mask_weight.reshape(-1),
        jnp.asarray(_FPAT), jnp.asarray(_BLPAT), jnp.asarray(_OFFPAT),
    )
    return jnp.transpose(out2d.reshape(_F, _D, _B), (2, 0, 1))


# speedup vs baseline: 4.8183x; 1.0532x over previous
import functools

import jax
import jax.numpy as jnp
import numpy as np
from jax import lax
from jax.experimental import pallas as pl
from jax.experimental.pallas import tpu as pltpu
from jax.experimental.pallas import tpu_sc as plsc

_CARD = 100000
_F = 26
_D = 32
_B = 16384
_T = 2600000
_NW = 32
_WB = _B // _NW
_CB = 64
_C = _CB * _F
_GB = 128
_NQ = _C // _GB
_NCH = _WB // _CB

_TQ = 512
_NQUAD = 20312 * 128 // _TQ
_G1 = 80
_REM = 64


def _build_patterns():
    e = np.arange(_C, dtype=np.int32)
    f = e % _F
    bl = e // _F
    return (
        f.reshape(_NQ, _GB),
        bl.reshape(_NQ, _GB),
        (f * _CARD).reshape(_NQ, _GB),
    )


_FPAT, _BLPAT, _OFFPAT = _build_patterns()


def _make_relayout():
    mesh = plsc.VectorSubcoreMesh(core_axis_name="c", subcore_axis_name="s")

    @functools.partial(
        pl.kernel,
        mesh=mesh,
        out_type=jax.ShapeDtypeStruct((_T // 4, 128), jnp.float32),
        compiler_params=pltpu.CompilerParams(
            use_tc_tiling_on_sc=True, needs_layout_passes=False
        ),
        scratch_types=[
            pltpu.VMEM((_D, _TQ + 1), jnp.float32),
            pltpu.VMEM((_D, _TQ + 1), jnp.float32),
            pltpu.VMEM((_TQ // 4, 128), jnp.float32),
            pltpu.SemaphoreType.DMA,
            pltpu.SemaphoreType.DMA,
        ],
    )
    def k1(tt_hbm, tail_hbm, t4_hbm, ca_v, cb_v, r_v, sa, sb):
        nc = 2
        wid = lax.axis_index("s") * nc + lax.axis_index("c")
        iota = lax.iota(jnp.int32, 16)
        c16lo = iota
        c16hi = iota + 16

        def fire(q4, buf, sem):
            pltpu.async_copy(
                tt_hbm.at[:, pl.ds(q4 * _TQ, _TQ)],
                buf.at[:, pl.ds(0, _TQ)],
                sem,
            )

        def process(g, buf, sem, nbuf, nsem):
            q4 = g * _NW + wid

            @pl.when(q4 < _NQUAD)
            def _():
                pltpu.make_async_copy(
                    tt_hbm.at[:, pl.ds(0, _TQ)], buf.at[:, pl.ds(0, _TQ)], sem
                ).wait()
                qn = (g + 1) * _NW + wid

                @pl.when(qn < _NQUAD)
                def _():
                    fire(qn, nbuf, nsem)

                @plsc.parallel_loop(0, _TQ // 4, unroll=2)
                def r_body(rr):
                    for h in range(8):
                        rloc = rr * 4 + h // 2
                        c16 = c16lo if h % 2 == 0 else c16hi
                        g16 = plsc.load_gather(
                            buf, [c16, jnp.broadcast_to(rloc, (16,))]
                        )
                        r_v[rr, pl.ds(h * 16, 16)] = g16

                pltpu.sync_copy(
                    r_v, t4_hbm.at[pl.ds(q4 * (_TQ // 4), _TQ // 4), :]
                )

        fire(wid, ca_v, sa)

        def pair_body(g2, carry):
            process(g2 * 2, ca_v, sa, cb_v, sb)
            process(g2 * 2 + 1, cb_v, sb, ca_v, sa)
            return carry

        lax.fori_loop(0, _G1, pair_body, 0)

        @pl.when(wid == 0)
        def _():
            pltpu.sync_copy(tail_hbm, r_v.at[pl.ds(0, _REM // 4), :])
            pltpu.sync_copy(r_v.at[pl.ds(0, _REM // 4), :],
                            t4_hbm.at[pl.ds(_NQUAD * (_TQ // 4), _REM // 4), :])

    return k1


def _make_gather():
    mesh = plsc.VectorSubcoreMesh(core_axis_name="c", subcore_axis_name="s")

    @functools.partial(
        pl.kernel,
        mesh=mesh,
        out_type=jax.ShapeDtypeStruct((_F * _D, _B), jnp.float32),
        compiler_params=pltpu.CompilerParams(
            use_tc_tiling_on_sc=False, needs_layout_passes=False
        ),
        scratch_types=[
            pltpu.VMEM((_F, _CB), jnp.int32),
            pltpu.VMEM((_F, _CB), jnp.int32),
            pltpu.VMEM((_NQ, _GB), jnp.int32),
            pltpu.VMEM((_C, _D), jnp.float32),
            pltpu.VMEM((_F * _D, _CB), jnp.float32),
            pltpu.VMEM((_NQ, _GB), jnp.int32),
            pltpu.VMEM((_NQ, _GB), jnp.int32),
            pltpu.VMEM((_NQ, _GB), jnp.int32),
            pltpu.VMEM((_F * _D,), jnp.float32),
            pltpu.VMEM((_D,), jnp.float32),
            pltpu.SemaphoreType.DMA,
            pltpu.SemaphoreType.DMA,
        ],
    )
    def k2(xt_hbm, mt_hbm, tl_hbm, bias_hbm, mw_hbm, fp_hbm, blp_hbm,
           op_hbm, out_hbm,
           x_v, m_v, idx_v, g_v, t_v, fp_v, blp_v, op_v, bias_v,
           mw_v, sem, so):
        nc = 2
        wid = lax.axis_index("s") * nc + lax.axis_index("c")
        b0 = wid * _WB
        pltpu.sync_copy(bias_hbm, bias_v)
        pltpu.sync_copy(mw_hbm, mw_v)
        pltpu.sync_copy(fp_hbm, fp_v)
        pltpu.sync_copy(blp_hbm, blp_v)
        pltpu.sync_copy(op_hbm, op_v)
        mw0 = mw_v[pl.ds(0, 16)]
        mw1 = mw_v[pl.ds(16, 16)]
        iota = lax.iota(jnp.int32, 16)

        def chunk_body(ch, carry):
            bc = b0 + ch * _CB
            pltpu.sync_copy(xt_hbm.at[:, pl.ds(bc, _CB)], x_v)
            pltpu.sync_copy(mt_hbm.at[:, pl.ds(bc, _CB)], m_v)

            @plsc.parallel_loop(0, _NQ)
            def idx_body(q):
                for kk in range(_GB // 16):
                    sl = pl.ds(kk * 16, 16)
                    f16 = fp_v[q, sl]
                    bl16 = blp_v[q, sl]
                    xv = plsc.load_gather(x_v, [f16, bl16])
                    idx_v[q, sl] = xv + op_v[q, sl]

            for q in range(_NQ):
                pltpu.async_copy(
                    tl_hbm.at[idx_v.at[q]],
                    g_v.at[pl.ds(q * _GB, _GB)],
                    sem,
                )
            pltpu.make_async_copy(
                tl_hbm.at[pl.ds(0, _C), :], g_v, sem
            ).wait()

            @pl.when(ch > 0)
            def _():
                pltpu.make_async_copy(
                    out_hbm.at[:, pl.ds(0, _CB)], t_v, so
                ).wait()

            @plsc.parallel_loop(0, _F)
            def f_body(f):
                bv0 = bias_v[pl.ds(f * _D, 16)]
                bv1 = bias_v[pl.ds(f * _D + 16, 16)]
                krow = [
                    jnp.float32(1)
                    - m_v[f, pl.ds(v * 16, 16)].astype(jnp.float32)
                    for v in range(_CB // 16)
                ]
                erow = [
                    (v * 16 + iota) * _F + f for v in range(_CB // 16)
                ]
                for c in range(_D):
                    a = (bv0 if c < 16 else bv1)[c % 16]
                    w = (mw0 if c < 16 else mw1)[c % 16]
                    p = a - w
                    c16 = jnp.broadcast_to(jnp.int32(c), (16,))
                    row = f * _D + c
                    for v in range(_CB // 16):
                        g16 = plsc.load_gather(g_v, [erow[v], c16])
                        t_v[row, pl.ds(v * 16, 16)] = (g16 + p) * krow[v] + w

            pltpu.async_copy(t_v, out_hbm.at[:, pl.ds(bc, _CB)], so)
            return carry

        lax.fori_loop(0, _NCH, chunk_body, 0)
        pltpu.make_async_copy(out_hbm.at[:, pl.ds(0, _CB)], t_v, so).wait()

    return k2


_K1 = _make_relayout()
_K2 = _make_gather()


@jax.jit
def kernel(x, mask, table, bias, mask_weight):
    tt = jnp.transpose(table)
    tail4 = table[_NQUAD * _TQ:].reshape(_REM // 4, 128)
    t4 = _K1(tt, tail4)
    tlin = t4.reshape(_T, _D)
    xt = jnp.transpose(x)
    mt = jnp.transpose(mask).astype(jnp.int32)
    out2d = _K2(
        xt, mt, tlin, bias.reshape(-1), mask_weight.reshape(-1),
        jnp.asarray(_FPAT), jnp.asarray(_BLPAT), jnp.asarray(_OFFPAT),
    )
    return jnp.transpose(out2d.reshape(_F, _D, _B), (2, 0, 1))
